# baseline (speedup 1.0000x reference)
import functools
import math

import jax
import jax.numpy as jnp
from jax import lax
from jax.experimental import pallas as pl
from jax.experimental.pallas import tpu as pltpu
from jax.experimental.pallas import tpu_sc as plsc

D = 128
SCALE = math.sqrt(float(D))

NC = 2
NS = 16
NW = NC * NS

B_TOTAL = 1024 * 200
B_PER_W = B_TOTAL // NW
CHUNK = 128
NCHUNK = B_PER_W // CHUNK


def _emb_body(x_hbm, table_hbm, out_hbm, idx_v,
              gbuf0, gbuf1, sbuf0, sbuf1,
              gsem0, gsem1, ssem0, ssem1):
    gbuf = (gbuf0, gbuf1)
    sbuf = (sbuf0, sbuf1)
    gsem = (gsem0, gsem1)
    ssem = (ssem0, ssem1)

    wid = lax.axis_index("s") * NC + lax.axis_index("c")
    base = wid * B_PER_W
    pltpu.sync_copy(x_hbm.at[pl.ds(base, B_PER_W)], idx_v)

    def g_start(j, b):
        off = pl.multiple_of(j * CHUNK, CHUNK)
        pltpu.async_copy(table_hbm.at[idx_v.at[pl.ds(off, CHUNK)]],
                         gbuf[b], gsem[b])

    def g_wait(b):
        pltpu.make_async_copy(table_hbm.at[idx_v.at[pl.ds(0, CHUNK)]],
                              gbuf[b], gsem[b]).wait()

    def s_start(j, b):
        off = pl.multiple_of(j * CHUNK, CHUNK)
        pltpu.async_copy(sbuf[b], out_hbm.at[pl.ds(base + off, CHUNK)],
                         ssem[b])

    def s_wait(b):
        pltpu.make_async_copy(sbuf[b], out_hbm.at[pl.ds(base, CHUNK)],
                              ssem[b]).wait()

    def scale(b):
        gb, sb = gbuf[b], sbuf[b]

        def srow(i, carry):
            r0 = i * 4
            for dr in range(4):
                r = r0 + dr
                for c in range(D // 16):
                    sl = pl.ds(c * 16, 16)
                    sb[r, sl] = gb[r, sl] * SCALE
            return carry

        lax.fori_loop(0, CHUNK // 4, srow, 0)

    g_start(0, 0)

    def body(i, carry):
        j0 = i * 2
        for b in range(2):
            j = j0 + b
            g_start(j + 1, 1 - b)
            g_wait(b)

            @pl.when(j >= 2)
            def _():
                s_wait(b)

            scale(b)
            s_start(j, b)
        return carry

    lax.fori_loop(0, (NCHUNK - 2) // 2, body, 0)

    g_start(NCHUNK - 1, 1)
    g_wait(0)
    s_wait(0)
    scale(0)
    s_start(NCHUNK - 2, 0)
    g_wait(1)
    s_wait(1)
    scale(1)
    s_start(NCHUNK - 1, 1)
    s_wait(0)
    s_wait(1)


_emb = functools.partial(
    pl.kernel,
    mesh=plsc.VectorSubcoreMesh(core_axis_name="c", subcore_axis_name="s"),
    out_type=jax.ShapeDtypeStruct((B_TOTAL, D), jnp.float32),
    scratch_types=[
        pltpu.VMEM((B_PER_W,), jnp.int32),
        pltpu.VMEM((CHUNK, D), jnp.float32),
        pltpu.VMEM((CHUNK, D), jnp.float32),
        pltpu.VMEM((CHUNK, D), jnp.float32),
        pltpu.VMEM((CHUNK, D), jnp.float32),
        pltpu.SemaphoreType.DMA,
        pltpu.SemaphoreType.DMA,
        pltpu.SemaphoreType.DMA,
        pltpu.SemaphoreType.DMA,
    ],
)(_emb_body)


def kernel(x, table):
    xf = x.reshape(-1).astype(jnp.int32)
    out = _emb(xf, table)
    return out.reshape(x.shape + (D,))


# speedup vs baseline: 1.0053x; 1.0053x over previous
import functools
import math

import jax
import jax.numpy as jnp
from jax import lax
from jax.experimental import pallas as pl
from jax.experimental.pallas import tpu as pltpu
from jax.experimental.pallas import tpu_sc as plsc

D = 128
SCALE = math.sqrt(float(D))

NC = 2
NS = 16
NW = NC * NS

B_TOTAL = 1024 * 200
B_PER_W = B_TOTAL // NW
CHUNK = 200
SPLITS = ((0, 128), (128, 72))
NCHUNK = B_PER_W // CHUNK
ROW_UNROLL = 4


def _emb_body(x_hbm, table_hbm, out_hbm, idx_v,
              gbuf0, gbuf1, sbuf0, sbuf1,
              gsem0, gsem1, ssem0, ssem1):
    gbuf = (gbuf0, gbuf1)
    sbuf = (sbuf0, sbuf1)
    gsem = (gsem0, gsem1)
    ssem = (ssem0, ssem1)

    wid = lax.axis_index("s") * NC + lax.axis_index("c")
    base = wid * B_PER_W
    pltpu.sync_copy(x_hbm.at[pl.ds(base, B_PER_W)], idx_v)

    def g_start(j, b):
        off = pl.multiple_of(j * CHUNK, 8)
        for lo, n in SPLITS:
            pltpu.async_copy(
                table_hbm.at[idx_v.at[pl.ds(off + lo, n)]],
                gbuf[b].at[pl.ds(lo, n)], gsem[b])

    def g_wait(b):
        for lo, n in SPLITS:
            pltpu.make_async_copy(
                table_hbm.at[idx_v.at[pl.ds(lo, n)]],
                gbuf[b].at[pl.ds(lo, n)], gsem[b]).wait()

    def s_start(j, b):
        off = pl.multiple_of(j * CHUNK, 8)
        pltpu.async_copy(sbuf[b], out_hbm.at[pl.ds(base + off, CHUNK)],
                         ssem[b])

    def s_wait(b):
        pltpu.make_async_copy(sbuf[b], out_hbm.at[pl.ds(base, CHUNK)],
                              ssem[b]).wait()

    def scale(b):
        gb, sb = gbuf[b], sbuf[b]

        def srow(i, carry):
            r0 = i * ROW_UNROLL
            for dr in range(ROW_UNROLL):
                r = r0 + dr
                for c in range(D // 16):
                    sl = pl.ds(c * 16, 16)
                    sb[r, sl] = gb[r, sl] * SCALE
            return carry

        lax.fori_loop(0, CHUNK // ROW_UNROLL, srow, 0)

    g_start(0, 0)

    def body(i, carry):
        j0 = i * 2
        for b in range(2):
            j = j0 + b
            g_start(j + 1, 1 - b)
            g_wait(b)

            @pl.when(j >= 2)
            def _():
                s_wait(b)

            scale(b)
            s_start(j, b)
        return carry

    lax.fori_loop(0, (NCHUNK - 2) // 2, body, 0)

    g_start(NCHUNK - 1, 1)
    g_wait(0)
    s_wait(0)
    scale(0)
    s_start(NCHUNK - 2, 0)
    g_wait(1)
    s_wait(1)
    scale(1)
    s_start(NCHUNK - 1, 1)
    s_wait(0)
    s_wait(1)


_emb = functools.partial(
    pl.kernel,
    mesh=plsc.VectorSubcoreMesh(core_axis_name="c", subcore_axis_name="s"),
    out_type=jax.ShapeDtypeStruct((B_TOTAL, D), jnp.float32),
    scratch_types=[
        pltpu.VMEM((B_PER_W,), jnp.int32),
        pltpu.VMEM((CHUNK, D), jnp.float32),
        pltpu.VMEM((CHUNK, D), jnp.float32),
        pltpu.VMEM((CHUNK, D), jnp.float32),
        pltpu.VMEM((CHUNK, D), jnp.float32),
        pltpu.SemaphoreType.DMA,
        pltpu.SemaphoreType.DMA,
        pltpu.SemaphoreType.DMA,
        pltpu.SemaphoreType.DMA,
    ],
)(_emb_body)


def kernel(x, table):
    xf = x.reshape(-1).astype(jnp.int32)
    out = _emb(xf, table)
    return out.reshape(x.shape + (D,))


# speedup vs baseline: 1.0107x; 1.0054x over previous
import functools
import math

import jax
import jax.numpy as jnp
from jax import lax
from jax.experimental import pallas as pl
from jax.experimental.pallas import tpu as pltpu
from jax.experimental.pallas import tpu_sc as plsc

D = 128
SCALE = math.sqrt(float(D))

NC = 2
NS = 16
NW = NC * NS

B_TOTAL = 1024 * 200
B_PER_W = B_TOTAL // NW
CHUNK = 128
NCHUNK = B_PER_W // CHUNK
NBUF = 3
ROW_UNROLL = 4


def _emb_body(x_hbm, table_hbm, out_hbm, idx_v,
              gbuf0, gbuf1, gbuf2, sbuf0, sbuf1, sbuf2,
              gsem0, gsem1, gsem2, ssem0, ssem1, ssem2):
    gbuf = (gbuf0, gbuf1, gbuf2)
    sbuf = (sbuf0, sbuf1, sbuf2)
    gsem = (gsem0, gsem1, gsem2)
    ssem = (ssem0, ssem1, ssem2)

    wid = lax.axis_index("s") * NC + lax.axis_index("c")
    base = wid * B_PER_W
    pltpu.sync_copy(x_hbm.at[pl.ds(base, B_PER_W)], idx_v)

    def g_start(j, b):
        off = pl.multiple_of(j * CHUNK, 8)
        pltpu.async_copy(table_hbm.at[idx_v.at[pl.ds(off, CHUNK)]],
                         gbuf[b], gsem[b])

    def g_wait(b):
        pltpu.make_async_copy(table_hbm.at[idx_v.at[pl.ds(0, CHUNK)]],
                              gbuf[b], gsem[b]).wait()

    def s_start(j, b):
        off = pl.multiple_of(j * CHUNK, 8)
        pltpu.async_copy(sbuf[b], out_hbm.at[pl.ds(base + off, CHUNK)],
                         ssem[b])

    def s_wait(b):
        pltpu.make_async_copy(sbuf[b], out_hbm.at[pl.ds(base, CHUNK)],
                              ssem[b]).wait()

    def scale(b):
        gb, sb = gbuf[b], sbuf[b]

        def srow(i, carry):
            r0 = i * ROW_UNROLL
            for dr in range(ROW_UNROLL):
                r = r0 + dr
                for c in range(D // 16):
                    sl = pl.ds(c * 16, 16)
                    sb[r, sl] = gb[r, sl] * SCALE
            return carry

        lax.fori_loop(0, CHUNK // ROW_UNROLL, srow, 0)

    g_start(0, 0)
    g_start(1, 1)

    def body(i, carry):
        j0 = i * NBUF
        for b in range(NBUF):
            j = j0 + b
            g_start(j + 2, (b + 2) % NBUF)
            g_wait(b)

            @pl.when(j >= NBUF)
            def _():
                s_wait(b)

            scale(b)
            s_start(j, b)
        return carry

    lax.fori_loop(0, (NCHUNK - 2) // NBUF, body, 0)

    g_wait(0)
    s_wait(0)
    scale(0)
    s_start(NCHUNK - 2, 0)
    g_wait(1)
    s_wait(1)
    scale(1)
    s_start(NCHUNK - 1, 1)
    s_wait(2)
    s_wait(0)
    s_wait(1)


_emb = functools.partial(
    pl.kernel,
    mesh=plsc.VectorSubcoreMesh(core_axis_name="c", subcore_axis_name="s"),
    out_type=jax.ShapeDtypeStruct((B_TOTAL, D), jnp.float32),
    scratch_types=[
        pltpu.VMEM((B_PER_W,), jnp.int32),
        pltpu.VMEM((CHUNK, D), jnp.float32),
        pltpu.VMEM((CHUNK, D), jnp.float32),
        pltpu.VMEM((CHUNK, D), jnp.float32),
        pltpu.VMEM((CHUNK, D), jnp.float32),
        pltpu.VMEM((CHUNK, D), jnp.float32),
        pltpu.VMEM((CHUNK, D), jnp.float32),
        pltpu.SemaphoreType.DMA,
        pltpu.SemaphoreType.DMA,
        pltpu.SemaphoreType.DMA,
        pltpu.SemaphoreType.DMA,
        pltpu.SemaphoreType.DMA,
        pltpu.SemaphoreType.DMA,
    ],
)(_emb_body)


def kernel(x, table):
    xf = x.reshape(-1).astype(jnp.int32)
    out = _emb(xf, table)
    return out.reshape(x.shape + (D,))
